# baseline (speedup 1.0000x reference)
import functools

import jax
import jax.numpy as jnp
from jax import lax
from jax.experimental import pallas as pl
from jax.experimental.pallas import tpu as pltpu

_NC = 4


def _u_chunk_copy(u_hbm, u_vmem, sems, base, j, cj):
    return pltpu.make_async_copy(
        u_hbm.at[:, pl.ds(base + j * cj, cj)],
        u_vmem.at[:, pl.ds(j * cj, cj)],
        sems.at[j],
    )


def _partial_kernel(ve_ref, la_ref, x_ref, w1_ref, u_hbm,
                    hp_ref,
                    u_vmem, t2_ref, sems):
    f32 = jnp.float32
    c = pl.program_id(0)
    nh = u_vmem.shape[1]
    cj = nh // _NC
    base = c * nh

    for j in range(_NC):
        _u_chunk_copy(u_hbm, u_vmem, sems, base, j, cj).start()

    X = x_ref[...]

    for j in range(_NC):
        _u_chunk_copy(u_hbm, u_vmem, sems, base, j, cj).wait()
        t2_ref[:, pl.ds(j * cj, cj)] = lax.dot_general(
            X, u_vmem[:, pl.ds(j * cj, cj)], (((0,), (0,)), ((), ())),
            preferred_element_type=f32)

    vla = jnp.power(la_ref[...], ve_ref[0])
    Tv2 = t2_ref[...] * vla

    Tw2 = lax.dot_general(w1_ref[...], Tv2, (((1,), (0,)), ((), ())),
                          preferred_element_type=f32)

    hp_ref[0] = lax.dot_general(u_vmem[...], Tw2, (((1,), (1,)), ((), ())),
                                preferred_element_type=f32)


def _combine_kernel(hp_ref, b1_ref, gamma_ref, beta_ref, w2_ref, b2_ref,
                    out_ref, hid_ref):
    f32 = jnp.float32
    H = hp_ref[0] + hp_ref[1] + b1_ref[...]
    hid_ref[...] = H

    mean = jnp.mean(H, axis=0, keepdims=True)
    var = jnp.mean(jnp.square(H - mean), axis=0, keepdims=True)
    Hn = (H - mean) * lax.rsqrt(var + 1e-5)
    Hn = Hn * gamma_ref[...] + beta_ref[...]

    Hr = jnp.maximum(Hn, 0.0)

    logits = lax.dot_general(Hr, w2_ref[...], (((1,), (1,)), ((), ())),
                             preferred_element_type=f32) + b2_ref[...]

    m = jnp.max(logits, axis=1, keepdims=True)
    z = logits - m
    lse = jnp.log(jnp.sum(jnp.exp(z), axis=1, keepdims=True))
    out_ref[...] = z - lse


def kernel(X, La, U, ve, W_w, W_b, bn_gamma, bn_beta, MLP_w, MLP_b):
    N, in_c = X.shape
    hidden = W_w.shape[0]
    out_c = MLP_w.shape[0]
    nh = N // 2

    vmem = pl.BlockSpec(memory_space=pltpu.MemorySpace.VMEM)
    smem = pl.BlockSpec(memory_space=pltpu.MemorySpace.SMEM)
    hbm = pl.BlockSpec(memory_space=pltpu.MemorySpace.HBM)

    hp = pl.pallas_call(
        _partial_kernel,
        grid=(2,),
        out_shape=jax.ShapeDtypeStruct((2, N, hidden), jnp.float32),
        in_specs=[
            smem,
            pl.BlockSpec((1, nh), lambda c: (0, c)),
            pl.BlockSpec((N, in_c), lambda c: (0, 0)),
            pl.BlockSpec((hidden, in_c), lambda c: (0, 0)),
            hbm,
        ],
        out_specs=pl.BlockSpec((1, N, hidden), lambda c: (c, 0, 0)),
        scratch_shapes=[
            pltpu.VMEM((N, nh), jnp.float32),
            pltpu.VMEM((in_c, nh), jnp.float32),
            pltpu.SemaphoreType.DMA((_NC,)),
        ],
        compiler_params=pltpu.CompilerParams(
            dimension_semantics=("parallel",)),
    )(
        ve.astype(jnp.float32).reshape(1),
        La.reshape(1, N).astype(jnp.float32),
        X.astype(jnp.float32),
        W_w.astype(jnp.float32),
        U.astype(jnp.float32),
    )

    out, hidden_emd = pl.pallas_call(
        _combine_kernel,
        out_shape=(
            jax.ShapeDtypeStruct((N, out_c), jnp.float32),
            jax.ShapeDtypeStruct((N, hidden), jnp.float32),
        ),
        in_specs=[vmem] * 6,
        out_specs=(vmem, vmem),
    )(
        hp,
        W_b.reshape(1, hidden).astype(jnp.float32),
        bn_gamma.reshape(1, hidden).astype(jnp.float32),
        bn_beta.reshape(1, hidden).astype(jnp.float32),
        MLP_w.astype(jnp.float32),
        MLP_b.reshape(1, out_c).astype(jnp.float32),
    )
    return out, hidden_emd


# speedup vs baseline: 1.2182x; 1.2182x over previous
import functools

import jax
import jax.numpy as jnp
from jax import lax
from jax.experimental import pallas as pl
from jax.experimental.pallas import tpu as pltpu

_NC = 4


def _u_chunk_copy(u_hbm, u_vmem, sems, base, j, cj):
    return pltpu.make_async_copy(
        u_hbm.at[:, pl.ds(base + j * cj, cj)],
        u_vmem.at[:, pl.ds(j * cj, cj)],
        sems.at[j],
    )


def _partial_kernel(ve_ref, la_ref, x_ref, w1_ref, u_ref,
                    hp_ref):
    f32 = jnp.float32
    X = x_ref[...]
    Uc = u_ref[...]

    T2 = lax.dot_general(X, Uc, (((0,), (0,)), ((), ())),
                         preferred_element_type=f32)

    vla = jnp.power(la_ref[...], ve_ref[0])
    Tv2 = T2 * vla

    Tw2 = lax.dot_general(w1_ref[...], Tv2, (((1,), (0,)), ((), ())),
                          preferred_element_type=f32)

    hp_ref[0] = lax.dot_general(Uc, Tw2, (((1,), (1,)), ((), ())),
                                preferred_element_type=f32)


def _combine_kernel(hp_ref, b1_ref, gamma_ref, beta_ref, w2_ref, b2_ref,
                    out_ref, hid_ref):
    f32 = jnp.float32
    H = hp_ref[0] + hp_ref[1] + b1_ref[...]
    hid_ref[...] = H

    mean = jnp.mean(H, axis=0, keepdims=True)
    var = jnp.mean(jnp.square(H - mean), axis=0, keepdims=True)
    Hn = (H - mean) * lax.rsqrt(var + 1e-5)
    Hn = Hn * gamma_ref[...] + beta_ref[...]

    Hr = jnp.maximum(Hn, 0.0)

    logits = lax.dot_general(Hr, w2_ref[...], (((1,), (1,)), ((), ())),
                             preferred_element_type=f32) + b2_ref[...]

    m = jnp.max(logits, axis=1, keepdims=True)
    z = logits - m
    lse = jnp.log(jnp.sum(jnp.exp(z), axis=1, keepdims=True))
    out_ref[...] = z - lse


def kernel(X, La, U, ve, W_w, W_b, bn_gamma, bn_beta, MLP_w, MLP_b):
    N, in_c = X.shape
    hidden = W_w.shape[0]
    out_c = MLP_w.shape[0]
    nh = N // 2

    vmem = pl.BlockSpec(memory_space=pltpu.MemorySpace.VMEM)
    smem = pl.BlockSpec(memory_space=pltpu.MemorySpace.SMEM)
    hbm = pl.BlockSpec(memory_space=pltpu.MemorySpace.HBM)

    hp = pl.pallas_call(
        _partial_kernel,
        grid=(2,),
        out_shape=jax.ShapeDtypeStruct((2, N, hidden), jnp.float32),
        in_specs=[
            smem,
            pl.BlockSpec((1, nh), lambda c: (0, c)),
            pl.BlockSpec((N, in_c), lambda c: (0, 0)),
            pl.BlockSpec((hidden, in_c), lambda c: (0, 0)),
            pl.BlockSpec((N, nh), lambda c: (0, c)),
        ],
        out_specs=pl.BlockSpec((1, N, hidden), lambda c: (c, 0, 0)),
        compiler_params=pltpu.CompilerParams(
            dimension_semantics=("parallel",)),
    )(
        ve.astype(jnp.float32).reshape(1),
        La.reshape(1, N).astype(jnp.float32),
        X.astype(jnp.float32),
        W_w.astype(jnp.float32),
        U.astype(jnp.float32),
    )

    out, hidden_emd = pl.pallas_call(
        _combine_kernel,
        out_shape=(
            jax.ShapeDtypeStruct((N, out_c), jnp.float32),
            jax.ShapeDtypeStruct((N, hidden), jnp.float32),
        ),
        in_specs=[vmem] * 6,
        out_specs=(vmem, vmem),
    )(
        hp,
        W_b.reshape(1, hidden).astype(jnp.float32),
        bn_gamma.reshape(1, hidden).astype(jnp.float32),
        bn_beta.reshape(1, hidden).astype(jnp.float32),
        MLP_w.astype(jnp.float32),
        MLP_b.reshape(1, out_c).astype(jnp.float32),
    )
    return out, hidden_emd
